# Initial kernel scaffold; baseline (speedup 1.0000x reference)
import jax
import jax.numpy as jnp
from jax.experimental import pallas as pl


def kernel(boxes, scores, k):
    raise NotImplementedError("write your pallas kernel here")


# speedup vs baseline: 4.9707x; 4.9707x over previous
import jax
import jax.numpy as jnp
import numpy as np
from jax.experimental import pallas as pl
from jax.experimental.pallas import tpu as pltpu

_N = 20000
_R = 160
_L = 128
_NP = _R * _L
_K = 1000
_KP = 1024
_NEG = -1e9
_SENT = -3.0e38

_BITS = [b if (b := 1 << i) < 2**31 else b - 2**32 for i in range(32)]
_SIGN = -(2**31)


def _t(m):
    n = m.shape[1]
    i0 = jax.lax.broadcasted_iota(jnp.int32, (n, n), 0)
    i1 = jax.lax.broadcasted_iota(jnp.int32, (n, n), 1)
    eye = (i0 == i1).astype(jnp.float32)
    return jax.lax.dot_general(
        eye, m, (((1,), (1,)), ((), ())), preferred_element_type=jnp.float32, precision=jax.lax.Precision.HIGHEST)


def _nms_body(s_ref, x1_ref, y1_ref, x2_ref, y2_ref, k_ref, out_ref,
              iou_ref, msk_ref, pos_ref):
    f32 = jnp.float32

    s = s_ref[...]
    sig = jax.nn.sigmoid(s)
    masked = jnp.where(sig >= 0.1, s, f32(_NEG))
    msk_ref[...] = masked

    ibits = jax.lax.bitcast_convert_type(masked, jnp.int32)
    skey = jnp.where(ibits < 0, ibits ^ jnp.int32(0x7FFFFFFF), ibits)

    sign = jnp.int32(_SIGN)
    uT = jnp.int32(0)
    for b in range(31, -1, -1):
        cand_u = uT | jnp.int32(_BITS[b])
        cand_s = cand_u ^ sign
        cnt = jnp.sum((skey >= cand_s).astype(jnp.int32))
        uT = jnp.where(cnt >= _K, cand_u, uT)
    T = uT ^ sign

    gt = skey > T
    eq = skey == T
    need = (_K - jnp.sum(gt.astype(jnp.int32))).astype(f32)

    li = jax.lax.broadcasted_iota(jnp.int32, (_L, _L), 0)
    lj = jax.lax.broadcasted_iota(jnp.int32, (_L, _L), 1)
    tri_u = (li <= lj).astype(f32)
    ri = jax.lax.broadcasted_iota(jnp.int32, (_R, _R), 0)
    rj = jax.lax.broadcasted_iota(jnp.int32, (_R, _R), 1)
    tri_l = (rj < ri).astype(f32)

    def excl_cumsum(m):
        incl = jnp.dot(m, tri_u, preferred_element_type=f32, precision=jax.lax.Precision.HIGHEST)
        rs = jnp.sum(m, axis=1, keepdims=True)
        off = jnp.dot(tri_l, rs, preferred_element_type=f32, precision=jax.lax.Precision.HIGHEST)
        return incl - m + off

    eqrank = excl_cumsum(eq.astype(f32))
    sel = jnp.logical_or(gt, jnp.logical_and(eq, eqrank < need))
    pos = excl_cumsum(sel.astype(f32))
    pos_ref[...] = jnp.where(sel, pos, f32(-1.0))

    col_kp_f = jax.lax.broadcasted_iota(jnp.int32, (_KP, 1), 0).astype(f32)
    lane_row_f = jax.lax.broadcasted_iota(jnp.int32, (1, _L), 1).astype(f32)
    zrow = jnp.zeros((2, _L), f32)

    def comp_body(i, acc):
        pos_row = pos_ref[pl.ds(i, 1), :]
        onehot = (col_kp_f == pos_row).astype(f32)
        idx_row = lane_row_f + i.astype(f32) * f32(_L)
        rows = jnp.concatenate(
            [msk_ref[pl.ds(i, 1), :], idx_row,
             x1_ref[pl.ds(i, 1), :], y1_ref[pl.ds(i, 1), :],
             x2_ref[pl.ds(i, 1), :], y2_ref[pl.ds(i, 1), :], zrow], axis=0)
        data = _t(rows)
        return acc + jnp.dot(onehot, data, preferred_element_type=f32, precision=jax.lax.Precision.HIGHEST)

    acc = jax.lax.fori_loop(0, _R, comp_body, jnp.zeros((_KP, 8), f32))

    row_kp_f = jax.lax.broadcasted_iota(jnp.int32, (_KP, 1), 0).astype(f32)
    validrow = row_kp_f < f32(_K)
    rawc = jnp.where(validrow, acc[:, 0:1], f32(_SENT))
    idxc = jnp.where(validrow, acc[:, 1:2], f32(1e6) + row_kp_f)
    rawr = _t(rawc)
    idxr = _t(idxc)
    r_row = jnp.sum(((rawc > rawr).astype(f32)
                     + jnp.logical_and(rawc == rawr, idxc < idxr).astype(f32)),
                    axis=0, keepdims=True)
    perm = (col_kp_f == r_row).astype(f32)
    accadj = jnp.concatenate([rawc, idxc, acc[:, 2:8]], axis=1)
    srt = jnp.dot(perm, accadj, preferred_element_type=f32, precision=jax.lax.Precision.HIGHEST)
    srt_t = _t(srt)

    x1c, y1c = srt[:, 2:3], srt[:, 3:4]
    x2c, y2c = srt[:, 4:5], srt[:, 5:6]
    x1r, y1r = srt_t[2:3, :], srt_t[3:4, :]
    x2r, y2r = srt_t[4:5, :], srt_t[5:6, :]
    areac = (x2c - x1c) * (y2c - y1c)
    arear = (x2r - x1r) * (y2r - y1r)
    iw = jnp.maximum(jnp.minimum(x2c, x2r) - jnp.maximum(x1c, x1r), f32(0.0))
    ih = jnp.maximum(jnp.minimum(y2c, y2r) - jnp.maximum(y1c, y1r), f32(0.0))
    inter = iw * ih
    union = areac + arear - inter
    iou_ref[...] = inter / jnp.maximum(union, f32(1e-8))

    lane_kp_i = jax.lax.broadcasted_iota(jnp.int32, (1, _KP), 1)
    lane_kp_f = lane_kp_i.astype(f32)
    raw_row = srt_t[0:1, :]
    kval = k_ref[0, 0].astype(f32)
    keep0 = jnp.logical_and(raw_row > f32(-1e8), lane_kp_f < kval).astype(f32)

    def nms_step(i, keep):
        cur = jnp.sum(keep * (lane_kp_i == i).astype(f32))
        row = iou_ref[pl.ds(i, 1), :]
        sup = jnp.logical_and(row > f32(0.5), lane_kp_i > i).astype(f32)
        return keep * (f32(1.0) - cur * sup)

    keep = jax.lax.fori_loop(0, _K, nms_step, keep0)
    keepc = _t(keep)

    sigc = jax.nn.sigmoid(srt[:, 0:1])
    out = jnp.concatenate([srt[:, 2:6], sigc, jnp.zeros((_KP, 3), f32)],
                          axis=1)
    out_ref[...] = out * keepc


def kernel(boxes, scores, k):
    f32 = jnp.float32
    pad = _NP - _N
    s2d = jnp.concatenate(
        [scores.astype(f32), jnp.full((pad,), _NEG, f32)]).reshape(_R, _L)
    bx = jnp.concatenate([boxes.astype(f32), jnp.zeros((pad, 4), f32)], axis=0)
    x1 = bx[:, 0].reshape(_R, _L)
    y1 = bx[:, 1].reshape(_R, _L)
    x2 = bx[:, 2].reshape(_R, _L)
    y2 = bx[:, 3].reshape(_R, _L)
    k2d = jnp.asarray(k, jnp.int32).reshape(1, 1)

    out = pl.pallas_call(
        _nms_body,
        out_shape=jax.ShapeDtypeStruct((_KP, 8), f32),
        in_specs=[
            pl.BlockSpec(memory_space=pltpu.VMEM),
            pl.BlockSpec(memory_space=pltpu.VMEM),
            pl.BlockSpec(memory_space=pltpu.VMEM),
            pl.BlockSpec(memory_space=pltpu.VMEM),
            pl.BlockSpec(memory_space=pltpu.VMEM),
            pl.BlockSpec(memory_space=pltpu.SMEM),
        ],
        out_specs=pl.BlockSpec(memory_space=pltpu.VMEM),
        scratch_shapes=[
            pltpu.VMEM((_KP, _KP), f32),
            pltpu.VMEM((_R, _L), f32),
            pltpu.VMEM((_R, _L), f32),
        ],
    )(s2d, x1, y1, x2, y2, k2d)
    return out[:_K, :5]


# speedup vs baseline: 4.9972x; 1.0053x over previous
import jax
import jax.numpy as jnp
import numpy as np
from jax.experimental import pallas as pl
from jax.experimental.pallas import tpu as pltpu

_N = 20000
_R = 160
_L = 128
_NP = _R * _L
_K = 1000
_KP = 1024
_NEG = -1e9
_SENT = -3.0e38

_BITS = [b if (b := 1 << i) < 2**31 else b - 2**32 for i in range(32)]
_SIGN = -(2**31)


def _t(m):
    n = m.shape[1]
    i0 = jax.lax.broadcasted_iota(jnp.int32, (n, n), 0)
    i1 = jax.lax.broadcasted_iota(jnp.int32, (n, n), 1)
    eye = (i0 == i1).astype(jnp.float32)
    return jax.lax.dot_general(
        eye, m, (((1,), (1,)), ((), ())), preferred_element_type=jnp.float32, precision=jax.lax.Precision.HIGHEST)


def _nms_body(s_ref, x1_ref, y1_ref, x2_ref, y2_ref, k_ref, out_ref,
              iou_ref, msk_ref, pos_ref, blk_ref):
    f32 = jnp.float32

    s = s_ref[...]
    sig = jax.nn.sigmoid(s)
    masked = jnp.where(sig >= 0.1, s, f32(_NEG))
    msk_ref[...] = masked

    ibits = jax.lax.bitcast_convert_type(masked, jnp.int32)
    skey = jnp.where(ibits < 0, ibits ^ jnp.int32(0x7FFFFFFF), ibits)

    sign = jnp.int32(_SIGN)
    uT = jnp.int32(0)
    for b in range(31, -1, -1):
        cand_u = uT | jnp.int32(_BITS[b])
        cand_s = cand_u ^ sign
        cnt = jnp.sum((skey >= cand_s).astype(jnp.int32))
        uT = jnp.where(cnt >= _K, cand_u, uT)
    T = uT ^ sign

    gt = skey > T
    eq = skey == T
    need = (_K - jnp.sum(gt.astype(jnp.int32))).astype(f32)

    li = jax.lax.broadcasted_iota(jnp.int32, (_L, _L), 0)
    lj = jax.lax.broadcasted_iota(jnp.int32, (_L, _L), 1)
    tri_u = (li <= lj).astype(f32)
    ri = jax.lax.broadcasted_iota(jnp.int32, (_R, _R), 0)
    rj = jax.lax.broadcasted_iota(jnp.int32, (_R, _R), 1)
    tri_l = (rj < ri).astype(f32)

    def excl_cumsum(m):
        incl = jnp.dot(m, tri_u, preferred_element_type=f32, precision=jax.lax.Precision.HIGHEST)
        rs = jnp.sum(m, axis=1, keepdims=True)
        off = jnp.dot(tri_l, rs, preferred_element_type=f32, precision=jax.lax.Precision.HIGHEST)
        return incl - m + off

    eqrank = excl_cumsum(eq.astype(f32))
    sel = jnp.logical_or(gt, jnp.logical_and(eq, eqrank < need))
    pos = excl_cumsum(sel.astype(f32))
    pos_ref[...] = jnp.where(sel, pos, f32(-1.0))

    col_kp_f = jax.lax.broadcasted_iota(jnp.int32, (_KP, 1), 0).astype(f32)
    lane_row_f = jax.lax.broadcasted_iota(jnp.int32, (1, _L), 1).astype(f32)
    zrow = jnp.zeros((2, _L), f32)

    def comp_body(i, acc):
        pos_row = pos_ref[pl.ds(i, 1), :]
        onehot = (col_kp_f == pos_row).astype(f32)
        idx_row = lane_row_f + i.astype(f32) * f32(_L)
        rows = jnp.concatenate(
            [msk_ref[pl.ds(i, 1), :], idx_row,
             x1_ref[pl.ds(i, 1), :], y1_ref[pl.ds(i, 1), :],
             x2_ref[pl.ds(i, 1), :], y2_ref[pl.ds(i, 1), :], zrow], axis=0)
        data = _t(rows)
        return acc + jnp.dot(onehot, data, preferred_element_type=f32, precision=jax.lax.Precision.HIGHEST)

    acc = jax.lax.fori_loop(0, _R, comp_body, jnp.zeros((_KP, 8), f32))

    row_kp_f = jax.lax.broadcasted_iota(jnp.int32, (_KP, 1), 0).astype(f32)
    validrow = row_kp_f < f32(_K)
    rawc = jnp.where(validrow, acc[:, 0:1], f32(_SENT))
    idxc = jnp.where(validrow, acc[:, 1:2], f32(1e6) + row_kp_f)
    rawr = _t(rawc)
    idxr = _t(idxc)
    r_row = jnp.sum(((rawc > rawr).astype(f32)
                     + jnp.logical_and(rawc == rawr, idxc < idxr).astype(f32)),
                    axis=0, keepdims=True)
    perm = (col_kp_f == r_row).astype(f32)
    accadj = jnp.concatenate([rawc, idxc, acc[:, 2:8]], axis=1)
    srt = jnp.dot(perm, accadj, preferred_element_type=f32, precision=jax.lax.Precision.HIGHEST)
    srt_t = _t(srt)

    x1c, y1c = srt[:, 2:3], srt[:, 3:4]
    x2c, y2c = srt[:, 4:5], srt[:, 5:6]
    x1r, y1r = srt_t[2:3, :], srt_t[3:4, :]
    x2r, y2r = srt_t[4:5, :], srt_t[5:6, :]
    areac = (x2c - x1c) * (y2c - y1c)
    arear = (x2r - x1r) * (y2r - y1r)
    iw = jnp.maximum(jnp.minimum(x2c, x2r) - jnp.maximum(x1c, x1r), f32(0.0))
    ih = jnp.maximum(jnp.minimum(y2c, y2r) - jnp.maximum(y1c, y1r), f32(0.0))
    inter = iw * ih
    union = areac + arear - inter
    iou_ref[...] = inter / jnp.maximum(union, f32(1e-8))

    lane_kp_i = jax.lax.broadcasted_iota(jnp.int32, (1, _KP), 1)
    lane_kp_f = lane_kp_i.astype(f32)
    raw_row = srt_t[0:1, :]
    kval = k_ref[0, 0].astype(f32)
    keep0 = jnp.logical_and(raw_row > f32(-1e8), lane_kp_f < kval).astype(f32)

    nb = _KP // _L
    lane_l_i = jax.lax.broadcasted_iota(jnp.int32, (1, _L), 1)
    blocks = []
    for b in range(nb):
        lo = b * _L
        kb = keep0[:, lo:lo + _L]
        if b > 0:
            prev = jnp.concatenate(
                blocks + [jnp.zeros((1, _KP - lo), f32)], axis=1)
            adj = (iou_ref[:, lo:lo + _L] > f32(0.5)).astype(f32)
            supc = jnp.dot(prev, adj, preferred_element_type=f32,
                           precision=jax.lax.Precision.HIGHEST)
            kb = kb * (supc == f32(0.0)).astype(f32)

        blk_ref[...] = iou_ref[lo:lo + _L, lo:lo + _L]

        def blk_step(i, kblk):
            cur = jnp.sum(kblk * (lane_l_i == i).astype(f32))
            row = blk_ref[pl.ds(i, 1), :]
            sup = jnp.logical_and(row > f32(0.5), lane_l_i > i).astype(f32)
            return kblk * (f32(1.0) - cur * sup)

        kb = jax.lax.fori_loop(0, _L, blk_step, kb)
        blocks.append(kb)
    keep = jnp.concatenate(blocks, axis=1)
    keepc = _t(keep)

    sigc = jax.nn.sigmoid(srt[:, 0:1])
    out = jnp.concatenate([srt[:, 2:6], sigc, jnp.zeros((_KP, 3), f32)],
                          axis=1)
    out_ref[...] = out * keepc


def kernel(boxes, scores, k):
    f32 = jnp.float32
    pad = _NP - _N
    s2d = jnp.concatenate(
        [scores.astype(f32), jnp.full((pad,), _NEG, f32)]).reshape(_R, _L)
    bx = jnp.concatenate([boxes.astype(f32), jnp.zeros((pad, 4), f32)], axis=0)
    x1 = bx[:, 0].reshape(_R, _L)
    y1 = bx[:, 1].reshape(_R, _L)
    x2 = bx[:, 2].reshape(_R, _L)
    y2 = bx[:, 3].reshape(_R, _L)
    k2d = jnp.asarray(k, jnp.int32).reshape(1, 1)

    out = pl.pallas_call(
        _nms_body,
        out_shape=jax.ShapeDtypeStruct((_KP, 8), f32),
        in_specs=[
            pl.BlockSpec(memory_space=pltpu.VMEM),
            pl.BlockSpec(memory_space=pltpu.VMEM),
            pl.BlockSpec(memory_space=pltpu.VMEM),
            pl.BlockSpec(memory_space=pltpu.VMEM),
            pl.BlockSpec(memory_space=pltpu.VMEM),
            pl.BlockSpec(memory_space=pltpu.SMEM),
        ],
        out_specs=pl.BlockSpec(memory_space=pltpu.VMEM),
        scratch_shapes=[
            pltpu.VMEM((_KP, _KP), f32),
            pltpu.VMEM((_R, _L), f32),
            pltpu.VMEM((_R, _L), f32),
            pltpu.VMEM((_L, _L), f32),
        ],
    )(s2d, x1, y1, x2, y2, k2d)
    return out[:_K, :5]


# speedup vs baseline: 13.1490x; 2.6313x over previous
import jax
import jax.numpy as jnp
import numpy as np
from jax.experimental import pallas as pl
from jax.experimental.pallas import tpu as pltpu

_N = 20000
_R = 160
_L = 128
_NP = _R * _L
_K = 1000
_KP = 1024
_NEG = -1e9
_SENT = -3.0e38

_BITS = [b if (b := 1 << i) < 2**31 else b - 2**32 for i in range(32)]
_SIGN = -(2**31)


def _t(m):
    n = m.shape[1]
    i0 = jax.lax.broadcasted_iota(jnp.int32, (n, n), 0)
    i1 = jax.lax.broadcasted_iota(jnp.int32, (n, n), 1)
    eye = (i0 == i1).astype(jnp.float32)
    return jax.lax.dot_general(
        eye, m, (((1,), (1,)), ((), ())), preferred_element_type=jnp.float32, precision=jax.lax.Precision.HIGHEST)


def _nms_body(s_ref, x1_ref, y1_ref, x2_ref, y2_ref, k_ref, out_ref,
              iou_ref, msk_ref, pos_ref):
    f32 = jnp.float32

    s = s_ref[...]
    sig = jax.nn.sigmoid(s)
    masked = jnp.where(sig >= 0.1, s, f32(_NEG))
    msk_ref[...] = masked

    ibits = jax.lax.bitcast_convert_type(masked, jnp.int32)
    skey = jnp.where(ibits < 0, ibits ^ jnp.int32(0x7FFFFFFF), ibits)

    sign = jnp.int32(_SIGN)
    uT = jnp.int32(0)
    for b in range(31, -1, -1):
        cand_u = uT | jnp.int32(_BITS[b])
        cand_s = cand_u ^ sign
        cnt = jnp.sum((skey >= cand_s).astype(jnp.int32))
        uT = jnp.where(cnt >= _K, cand_u, uT)
    T = uT ^ sign

    gt = skey > T
    eq = skey == T
    need = (_K - jnp.sum(gt.astype(jnp.int32))).astype(f32)

    li = jax.lax.broadcasted_iota(jnp.int32, (_L, _L), 0)
    lj = jax.lax.broadcasted_iota(jnp.int32, (_L, _L), 1)
    tri_u = (li <= lj).astype(f32)
    ri = jax.lax.broadcasted_iota(jnp.int32, (_R, _R), 0)
    rj = jax.lax.broadcasted_iota(jnp.int32, (_R, _R), 1)
    tri_l = (rj < ri).astype(f32)

    def excl_cumsum(m):
        incl = jnp.dot(m, tri_u, preferred_element_type=f32, precision=jax.lax.Precision.HIGHEST)
        rs = jnp.sum(m, axis=1, keepdims=True)
        off = jnp.dot(tri_l, rs, preferred_element_type=f32, precision=jax.lax.Precision.HIGHEST)
        return incl - m + off

    eqrank = excl_cumsum(eq.astype(f32))
    sel = jnp.logical_or(gt, jnp.logical_and(eq, eqrank < need))
    pos = excl_cumsum(sel.astype(f32))
    pos_ref[...] = jnp.where(sel, pos, f32(-1.0))

    col_kp_f = jax.lax.broadcasted_iota(jnp.int32, (_KP, 1), 0).astype(f32)
    lane_kp_row = jax.lax.broadcasted_iota(jnp.int32, (1, _KP), 1).astype(f32)
    lane_row_f = jax.lax.broadcasted_iota(jnp.int32, (1, _L), 1).astype(f32)
    zrow = jnp.zeros((2, _L), f32)

    def comp_body(i, accT):
        pos_row = pos_ref[pl.ds(i, 1), :]
        pos_col = _t(pos_row)
        onehotT = (pos_col == lane_kp_row).astype(f32)
        idx_row = lane_row_f + i.astype(f32) * f32(_L)
        rows = jnp.concatenate(
            [msk_ref[pl.ds(i, 1), :], idx_row,
             x1_ref[pl.ds(i, 1), :], y1_ref[pl.ds(i, 1), :],
             x2_ref[pl.ds(i, 1), :], y2_ref[pl.ds(i, 1), :], zrow], axis=0)
        return accT + jnp.dot(rows, onehotT, preferred_element_type=f32, precision=jax.lax.Precision.HIGHEST)

    accT = jax.lax.fori_loop(0, _R, comp_body, jnp.zeros((8, _KP), f32))
    acc = _t(accT)

    row_kp_f = jax.lax.broadcasted_iota(jnp.int32, (_KP, 1), 0).astype(f32)
    validrow = row_kp_f < f32(_K)
    rawc = jnp.where(validrow, acc[:, 0:1], f32(_SENT))
    idxc = jnp.where(validrow, acc[:, 1:2], f32(1e6) + row_kp_f)
    rawr = _t(rawc)
    idxr = _t(idxc)
    r_row = jnp.sum(((rawc > rawr).astype(f32)
                     + jnp.logical_and(rawc == rawr, idxc < idxr).astype(f32)),
                    axis=0, keepdims=True)
    perm = (col_kp_f == r_row).astype(f32)
    accadj = jnp.concatenate([rawc, idxc, acc[:, 2:8]], axis=1)
    srt = jnp.dot(perm, accadj, preferred_element_type=f32, precision=jax.lax.Precision.HIGHEST)
    srt_t = _t(srt)

    x1c, y1c = srt[:, 2:3], srt[:, 3:4]
    x2c, y2c = srt[:, 4:5], srt[:, 5:6]
    x1r, y1r = srt_t[2:3, :], srt_t[3:4, :]
    x2r, y2r = srt_t[4:5, :], srt_t[5:6, :]
    areac = (x2c - x1c) * (y2c - y1c)
    arear = (x2r - x1r) * (y2r - y1r)
    iw = jnp.maximum(jnp.minimum(x2c, x2r) - jnp.maximum(x1c, x1r), f32(0.0))
    ih = jnp.maximum(jnp.minimum(y2c, y2r) - jnp.maximum(y1c, y1r), f32(0.0))
    inter = iw * ih
    union = areac + arear - inter
    iou_ref[...] = inter / jnp.maximum(union, f32(1e-8))

    lane_kp_i = jax.lax.broadcasted_iota(jnp.int32, (1, _KP), 1)
    lane_kp_f = lane_kp_i.astype(f32)
    raw_row = srt_t[0:1, :]
    kval = k_ref[0, 0].astype(f32)
    keep0 = jnp.logical_and(raw_row > f32(-1e8), lane_kp_f < kval).astype(f32)

    nb = _KP // _L
    blocks = []
    for b in range(nb):
        lo = b * _L
        kb = keep0[:, lo:lo + _L]
        if b > 0:
            prev = jnp.concatenate(
                blocks + [jnp.zeros((1, _KP - lo), f32)], axis=1)
            adj = (iou_ref[:, lo:lo + _L] > f32(0.5)).astype(f32)
            supc = jnp.dot(prev, adj, preferred_element_type=f32,
                           precision=jax.lax.Precision.HIGHEST)
            kb = kb * (supc == f32(0.0)).astype(f32)

        blk = iou_ref[lo:lo + _L, lo:lo + _L]
        a_ut = jnp.logical_and(blk > f32(0.5), li < lj).astype(f32)
        valid_b = kb

        def fp_cond(c):
            return c[1]

        def fp_body(c):
            x, _ = c
            supn = jnp.dot(x, a_ut, preferred_element_type=f32,
                           precision=jax.lax.Precision.HIGHEST)
            new = valid_b * (supn == f32(0.0)).astype(f32)
            return (new, jnp.any(new != x))

        kb, _ = jax.lax.while_loop(fp_cond, fp_body, (kb, True))
        blocks.append(kb)
    keep = jnp.concatenate(blocks, axis=1)
    keepc = _t(keep)

    sigc = jax.nn.sigmoid(srt[:, 0:1])
    out = jnp.concatenate([srt[:, 2:6], sigc, jnp.zeros((_KP, 3), f32)],
                          axis=1)
    out_ref[...] = out * keepc


def kernel(boxes, scores, k):
    f32 = jnp.float32
    pad = _NP - _N
    s2d = jnp.concatenate(
        [scores.astype(f32), jnp.full((pad,), _NEG, f32)]).reshape(_R, _L)
    bx = jnp.concatenate([boxes.astype(f32), jnp.zeros((pad, 4), f32)], axis=0)
    x1 = bx[:, 0].reshape(_R, _L)
    y1 = bx[:, 1].reshape(_R, _L)
    x2 = bx[:, 2].reshape(_R, _L)
    y2 = bx[:, 3].reshape(_R, _L)
    k2d = jnp.asarray(k, jnp.int32).reshape(1, 1)

    out = pl.pallas_call(
        _nms_body,
        out_shape=jax.ShapeDtypeStruct((_KP, 8), f32),
        in_specs=[
            pl.BlockSpec(memory_space=pltpu.VMEM),
            pl.BlockSpec(memory_space=pltpu.VMEM),
            pl.BlockSpec(memory_space=pltpu.VMEM),
            pl.BlockSpec(memory_space=pltpu.VMEM),
            pl.BlockSpec(memory_space=pltpu.VMEM),
            pl.BlockSpec(memory_space=pltpu.SMEM),
        ],
        out_specs=pl.BlockSpec(memory_space=pltpu.VMEM),
        scratch_shapes=[
            pltpu.VMEM((_KP, _KP), f32),
            pltpu.VMEM((_R, _L), f32),
            pltpu.VMEM((_R, _L), f32),
        ],
    )(s2d, x1, y1, x2, y2, k2d)
    return out[:_K, :5]


# speedup vs baseline: 13.3663x; 1.0165x over previous
import jax
import jax.numpy as jnp
import numpy as np
from jax.experimental import pallas as pl
from jax.experimental.pallas import tpu as pltpu

_N = 20000
_R = 160
_L = 128
_NP = _R * _L
_K = 1000
_KP = 1024
_NEG = -1e9
_SENT = -3.0e38

_BITS = [b if (b := 1 << i) < 2**31 else b - 2**32 for i in range(32)]
_SIGN = -(2**31)


def _t(m):
    n = m.shape[1]
    i0 = jax.lax.broadcasted_iota(jnp.int32, (n, n), 0)
    i1 = jax.lax.broadcasted_iota(jnp.int32, (n, n), 1)
    eye = (i0 == i1).astype(jnp.float32)
    return jax.lax.dot_general(
        eye, m, (((1,), (1,)), ((), ())), preferred_element_type=jnp.float32, precision=jax.lax.Precision.HIGHEST)


def _nms_body(s_ref, x1_ref, y1_ref, x2_ref, y2_ref, k_ref, out_ref,
              iou_ref, msk_ref, pos_ref):
    f32 = jnp.float32

    s = s_ref[...]
    sig = jax.nn.sigmoid(s)
    masked = jnp.where(sig >= 0.1, s, f32(_NEG))
    msk_ref[...] = masked

    ibits = jax.lax.bitcast_convert_type(masked, jnp.int32)
    skey = jnp.where(ibits < 0, ibits ^ jnp.int32(0x7FFFFFFF), ibits)

    sign = jnp.int32(_SIGN)
    uT = jnp.int32(0)
    for b in range(31, -1, -1):
        cand_u = uT | jnp.int32(_BITS[b])
        cand_s = cand_u ^ sign
        cnt = jnp.sum((skey >= cand_s).astype(jnp.int32))
        uT = jnp.where(cnt >= _K, cand_u, uT)
    T = uT ^ sign

    gt = skey > T
    eq = skey == T
    need = (_K - jnp.sum(gt.astype(jnp.int32))).astype(f32)

    li = jax.lax.broadcasted_iota(jnp.int32, (_L, _L), 0)
    lj = jax.lax.broadcasted_iota(jnp.int32, (_L, _L), 1)
    tri_u = (li <= lj).astype(f32)
    ri = jax.lax.broadcasted_iota(jnp.int32, (_R, _R), 0)
    rj = jax.lax.broadcasted_iota(jnp.int32, (_R, _R), 1)
    tri_l = (rj < ri).astype(f32)

    def excl_cumsum(m):
        incl = jnp.dot(m, tri_u, preferred_element_type=f32, precision=jax.lax.Precision.HIGHEST)
        rs = jnp.sum(m, axis=1, keepdims=True)
        off = jnp.dot(tri_l, rs, preferred_element_type=f32, precision=jax.lax.Precision.HIGHEST)
        return incl - m + off

    eqrank = excl_cumsum(eq.astype(f32))
    sel = jnp.logical_or(gt, jnp.logical_and(eq, eqrank < need))
    pos = excl_cumsum(sel.astype(f32))
    pos_ref[...] = jnp.where(sel, pos, f32(-1.0))

    col_kp_f = jax.lax.broadcasted_iota(jnp.int32, (_KP, 1), 0).astype(f32)
    lane_kp_row = jax.lax.broadcasted_iota(jnp.int32, (1, _KP), 1).astype(f32)
    lane_row_f = jax.lax.broadcasted_iota(jnp.int32, (1, _L), 1).astype(f32)
    zrow = jnp.zeros((2, _L), f32)

    def comp_one(i, accT):
        pos_row = pos_ref[pl.ds(i, 1), :]
        onehot = (col_kp_f == pos_row).astype(f32)
        idx_row = lane_row_f + i.astype(f32) * f32(_L)
        rows = jnp.concatenate(
            [msk_ref[pl.ds(i, 1), :], idx_row,
             x1_ref[pl.ds(i, 1), :], y1_ref[pl.ds(i, 1), :],
             x2_ref[pl.ds(i, 1), :], y2_ref[pl.ds(i, 1), :], zrow], axis=0)
        contrib = jax.lax.dot_general(
            rows, onehot, (((1,), (1,)), ((), ())),
            preferred_element_type=f32, precision=jax.lax.Precision.HIGHEST)
        return accT + contrib

    def comp_body(i, accT):
        base = i * 4
        for u in range(4):
            accT = comp_one(base + u, accT)
        return accT

    accT = jax.lax.fori_loop(0, _R // 4, comp_body, jnp.zeros((8, _KP), f32))
    acc = _t(accT)

    row_kp_f = jax.lax.broadcasted_iota(jnp.int32, (_KP, 1), 0).astype(f32)
    validrow = row_kp_f < f32(_K)
    rawc = jnp.where(validrow, acc[:, 0:1], f32(_SENT))
    idxc = jnp.where(validrow, acc[:, 1:2], f32(1e6) + row_kp_f)
    rawr = _t(rawc)
    idxr = _t(idxc)
    r_row = jnp.sum(((rawc > rawr).astype(f32)
                     + jnp.logical_and(rawc == rawr, idxc < idxr).astype(f32)),
                    axis=0, keepdims=True)
    perm = (col_kp_f == r_row).astype(f32)
    accadj = jnp.concatenate([rawc, idxc, acc[:, 2:8]], axis=1)
    srt = jnp.dot(perm, accadj, preferred_element_type=f32, precision=jax.lax.Precision.HIGHEST)
    srt_t = _t(srt)

    x1c, y1c = srt[:, 2:3], srt[:, 3:4]
    x2c, y2c = srt[:, 4:5], srt[:, 5:6]
    x1r, y1r = srt_t[2:3, :], srt_t[3:4, :]
    x2r, y2r = srt_t[4:5, :], srt_t[5:6, :]
    areac = (x2c - x1c) * (y2c - y1c)
    arear = (x2r - x1r) * (y2r - y1r)
    iw = jnp.maximum(jnp.minimum(x2c, x2r) - jnp.maximum(x1c, x1r), f32(0.0))
    ih = jnp.maximum(jnp.minimum(y2c, y2r) - jnp.maximum(y1c, y1r), f32(0.0))
    inter = iw * ih
    union = areac + arear - inter
    iou_ref[...] = inter / jnp.maximum(union, f32(1e-8))

    lane_kp_i = jax.lax.broadcasted_iota(jnp.int32, (1, _KP), 1)
    lane_kp_f = lane_kp_i.astype(f32)
    raw_row = srt_t[0:1, :]
    kval = k_ref[0, 0].astype(f32)
    keep0 = jnp.logical_and(raw_row > f32(-1e8), lane_kp_f < kval).astype(f32)

    nb = _KP // _L
    blocks = []
    for b in range(nb):
        lo = b * _L
        kb = keep0[:, lo:lo + _L]
        if b > 0:
            prev = jnp.concatenate(
                blocks + [jnp.zeros((1, _KP - lo), f32)], axis=1)
            adj = (iou_ref[:, lo:lo + _L] > f32(0.5)).astype(f32)
            supc = jnp.dot(prev, adj, preferred_element_type=f32,
                           precision=jax.lax.Precision.HIGHEST)
            kb = kb * (supc == f32(0.0)).astype(f32)

        blk = iou_ref[lo:lo + _L, lo:lo + _L]
        a_ut = jnp.logical_and(blk > f32(0.5), li < lj).astype(f32)
        valid_b = kb

        def fp_cond(c):
            return c[1]

        def fp_body(c):
            x, _ = c
            supn = jnp.dot(x, a_ut, preferred_element_type=f32,
                           precision=jax.lax.Precision.HIGHEST)
            new = valid_b * (supn == f32(0.0)).astype(f32)
            return (new, jnp.any(new != x))

        kb, _ = jax.lax.while_loop(fp_cond, fp_body, (kb, True))
        blocks.append(kb)
    keep = jnp.concatenate(blocks, axis=1)
    keepc = _t(keep)

    sigc = jax.nn.sigmoid(srt[:, 0:1])
    out = jnp.concatenate([srt[:, 2:6], sigc, jnp.zeros((_KP, 3), f32)],
                          axis=1)
    out_ref[...] = out * keepc


def kernel(boxes, scores, k):
    f32 = jnp.float32
    pad = _NP - _N
    s2d = jnp.concatenate(
        [scores.astype(f32), jnp.full((pad,), _NEG, f32)]).reshape(_R, _L)
    bx = jnp.concatenate([boxes.astype(f32), jnp.zeros((pad, 4), f32)], axis=0)
    x1 = bx[:, 0].reshape(_R, _L)
    y1 = bx[:, 1].reshape(_R, _L)
    x2 = bx[:, 2].reshape(_R, _L)
    y2 = bx[:, 3].reshape(_R, _L)
    k2d = jnp.asarray(k, jnp.int32).reshape(1, 1)

    out = pl.pallas_call(
        _nms_body,
        out_shape=jax.ShapeDtypeStruct((_KP, 8), f32),
        in_specs=[
            pl.BlockSpec(memory_space=pltpu.VMEM),
            pl.BlockSpec(memory_space=pltpu.VMEM),
            pl.BlockSpec(memory_space=pltpu.VMEM),
            pl.BlockSpec(memory_space=pltpu.VMEM),
            pl.BlockSpec(memory_space=pltpu.VMEM),
            pl.BlockSpec(memory_space=pltpu.SMEM),
        ],
        out_specs=pl.BlockSpec(memory_space=pltpu.VMEM),
        scratch_shapes=[
            pltpu.VMEM((_KP, _KP), f32),
            pltpu.VMEM((_R, _L), f32),
            pltpu.VMEM((_R, _L), f32),
        ],
    )(s2d, x1, y1, x2, y2, k2d)
    return out[:_K, :5]


# speedup vs baseline: 23.1900x; 1.7350x over previous
import jax
import jax.numpy as jnp
import numpy as np
from jax.experimental import pallas as pl
from jax.experimental.pallas import tpu as pltpu

_N = 20000
_R = 160
_L = 128
_NP = _R * _L
_K = 1000
_KP = 1024
_NEG = -1e9
_SENT = -3.0e38

_BITS = [b if (b := 1 << i) < 2**31 else b - 2**32 for i in range(32)]
_SIGN = -(2**31)


def _t(m):
    n = m.shape[1]
    i0 = jax.lax.broadcasted_iota(jnp.int32, (n, n), 0)
    i1 = jax.lax.broadcasted_iota(jnp.int32, (n, n), 1)
    eye = (i0 == i1).astype(jnp.float32)
    return jax.lax.dot_general(
        eye, m, (((1,), (1,)), ((), ())), preferred_element_type=jnp.float32, precision=jax.lax.Precision.HIGHEST)


def _nms_body(s_ref, x1_ref, y1_ref, x2_ref, y2_ref, k_ref, out_ref,
              iou_ref, msk_ref, pos_ref, off_ref):
    f32 = jnp.float32

    s = s_ref[...]
    sig = jax.nn.sigmoid(s)
    masked = jnp.where(sig >= 0.1, s, f32(_NEG))
    msk_ref[...] = masked

    ibits = jax.lax.bitcast_convert_type(masked, jnp.int32)
    skey = jnp.where(ibits < 0, ibits ^ jnp.int32(0x7FFFFFFF), ibits)

    sign = jnp.int32(_SIGN)
    uT = jnp.int32(0)
    for b in range(31, -1, -1):
        cand_u = uT | jnp.int32(_BITS[b])
        cand_s = cand_u ^ sign
        cnt = jnp.sum((skey >= cand_s).astype(jnp.int32))
        uT = jnp.where(cnt >= _K, cand_u, uT)
    T = uT ^ sign

    gt = skey > T
    eq = skey == T
    need = (_K - jnp.sum(gt.astype(jnp.int32))).astype(f32)

    li = jax.lax.broadcasted_iota(jnp.int32, (_L, _L), 0)
    lj = jax.lax.broadcasted_iota(jnp.int32, (_L, _L), 1)
    tri_u = (li <= lj).astype(f32)
    ri = jax.lax.broadcasted_iota(jnp.int32, (_R, _R), 0)
    rj = jax.lax.broadcasted_iota(jnp.int32, (_R, _R), 1)
    tri_l = (rj < ri).astype(f32)

    def excl_cumsum(m):
        incl = jnp.dot(m, tri_u, preferred_element_type=f32, precision=jax.lax.Precision.HIGHEST)
        rs = jnp.sum(m, axis=1, keepdims=True)
        off = jnp.dot(tri_l, rs, preferred_element_type=f32, precision=jax.lax.Precision.HIGHEST)
        return incl - m + off, off

    eqrank, _ = excl_cumsum(eq.astype(f32))
    sel = jnp.logical_or(gt, jnp.logical_and(eq, eqrank < need))
    pos, off = excl_cumsum(sel.astype(f32))
    pos_ref[...] = jnp.where(sel, pos, f32(-1.0))
    off_ref[...] = off

    col_kp_f = jax.lax.broadcasted_iota(jnp.int32, (_KP, 1), 0).astype(f32)
    lane_kp_row = jax.lax.broadcasted_iota(jnp.int32, (1, _KP), 1).astype(f32)
    lane_row_f = jax.lax.broadcasted_iota(jnp.int32, (1, _L), 1).astype(f32)
    zrow = jnp.zeros((2, _L), f32)

    col_l_f = jax.lax.broadcasted_iota(jnp.int32, (_L, 1), 0).astype(f32)
    zpad = jnp.zeros((8, _KP), f32)

    def comp_one(i, accT):
        start = off_ref[pl.ds(i, 1), :]
        relpos = pos_ref[pl.ds(i, 1), :] - start[0, 0]
        onehot = (col_l_f == relpos).astype(f32)
        idx_row = lane_row_f + i.astype(f32) * f32(_L)
        rows = jnp.concatenate(
            [msk_ref[pl.ds(i, 1), :], idx_row,
             x1_ref[pl.ds(i, 1), :], y1_ref[pl.ds(i, 1), :],
             x2_ref[pl.ds(i, 1), :], y2_ref[pl.ds(i, 1), :], zrow], axis=0)
        contrib = jax.lax.dot_general(
            rows, onehot, (((1,), (1,)), ((), ())),
            preferred_element_type=f32, precision=jax.lax.Precision.HIGHEST)
        padded = jnp.concatenate([contrib, zpad], axis=1)
        rolled = pltpu.roll(padded, start[0, 0].astype(jnp.int32), axis=1)
        return accT + rolled

    def comp_body(i, accT):
        base = i * 4
        for u in range(4):
            accT = comp_one(base + u, accT)
        return accT

    accT = jax.lax.fori_loop(0, _R // 4, comp_body,
                             jnp.zeros((8, _KP + _L), f32))
    acc = _t(accT[:, :_KP])

    row_kp_f = jax.lax.broadcasted_iota(jnp.int32, (_KP, 1), 0).astype(f32)
    validrow = row_kp_f < f32(_K)
    rawc = jnp.where(validrow, acc[:, 0:1], f32(_SENT))
    idxc = jnp.where(validrow, acc[:, 1:2], f32(1e6) + row_kp_f)
    rawr = _t(rawc)
    idxr = _t(idxc)
    r_row = jnp.sum(((rawc > rawr).astype(f32)
                     + jnp.logical_and(rawc == rawr, idxc < idxr).astype(f32)),
                    axis=0, keepdims=True)
    perm = (col_kp_f == r_row).astype(f32)
    accadj = jnp.concatenate([rawc, idxc, acc[:, 2:8]], axis=1)
    srt = jnp.dot(perm, accadj, preferred_element_type=f32, precision=jax.lax.Precision.HIGHEST)
    srt_t = _t(srt)

    x1c, y1c = srt[:, 2:3], srt[:, 3:4]
    x2c, y2c = srt[:, 4:5], srt[:, 5:6]
    x1r, y1r = srt_t[2:3, :], srt_t[3:4, :]
    x2r, y2r = srt_t[4:5, :], srt_t[5:6, :]
    areac = (x2c - x1c) * (y2c - y1c)
    arear = (x2r - x1r) * (y2r - y1r)
    iw = jnp.maximum(jnp.minimum(x2c, x2r) - jnp.maximum(x1c, x1r), f32(0.0))
    ih = jnp.maximum(jnp.minimum(y2c, y2r) - jnp.maximum(y1c, y1r), f32(0.0))
    inter = iw * ih
    union = areac + arear - inter
    iou_ref[...] = inter / jnp.maximum(union, f32(1e-8))

    lane_kp_i = jax.lax.broadcasted_iota(jnp.int32, (1, _KP), 1)
    lane_kp_f = lane_kp_i.astype(f32)
    raw_row = srt_t[0:1, :]
    kval = k_ref[0, 0].astype(f32)
    keep0 = jnp.logical_and(raw_row > f32(-1e8), lane_kp_f < kval).astype(f32)

    nb = _KP // _L
    blocks = []
    for b in range(nb):
        lo = b * _L
        kb = keep0[:, lo:lo + _L]
        if b > 0:
            prev = jnp.concatenate(
                blocks + [jnp.zeros((1, _KP - lo), f32)], axis=1)
            adj = (iou_ref[:, lo:lo + _L] > f32(0.5)).astype(f32)
            supc = jnp.dot(prev, adj, preferred_element_type=f32,
                           precision=jax.lax.Precision.HIGHEST)
            kb = kb * (supc == f32(0.0)).astype(f32)

        blk = iou_ref[lo:lo + _L, lo:lo + _L]
        a_ut = jnp.logical_and(blk > f32(0.5), li < lj).astype(f32)
        valid_b = kb

        def fp_cond(c):
            return c[1]

        def fp_body(c):
            x, _ = c
            supn = jnp.dot(x, a_ut, preferred_element_type=f32,
                           precision=jax.lax.Precision.HIGHEST)
            new = valid_b * (supn == f32(0.0)).astype(f32)
            return (new, jnp.any(new != x))

        kb, _ = jax.lax.while_loop(fp_cond, fp_body, (kb, True))
        blocks.append(kb)
    keep = jnp.concatenate(blocks, axis=1)
    keepc = _t(keep)

    sigc = jax.nn.sigmoid(srt[:, 0:1])
    out = jnp.concatenate([srt[:, 2:6], sigc, jnp.zeros((_KP, 3), f32)],
                          axis=1)
    out_ref[...] = out * keepc


def kernel(boxes, scores, k):
    f32 = jnp.float32
    pad = _NP - _N
    s2d = jnp.concatenate(
        [scores.astype(f32), jnp.full((pad,), _NEG, f32)]).reshape(_R, _L)
    bx = jnp.concatenate([boxes.astype(f32), jnp.zeros((pad, 4), f32)], axis=0)
    x1 = bx[:, 0].reshape(_R, _L)
    y1 = bx[:, 1].reshape(_R, _L)
    x2 = bx[:, 2].reshape(_R, _L)
    y2 = bx[:, 3].reshape(_R, _L)
    k2d = jnp.asarray(k, jnp.int32).reshape(1, 1)

    out = pl.pallas_call(
        _nms_body,
        out_shape=jax.ShapeDtypeStruct((_KP, 8), f32),
        in_specs=[
            pl.BlockSpec(memory_space=pltpu.VMEM),
            pl.BlockSpec(memory_space=pltpu.VMEM),
            pl.BlockSpec(memory_space=pltpu.VMEM),
            pl.BlockSpec(memory_space=pltpu.VMEM),
            pl.BlockSpec(memory_space=pltpu.VMEM),
            pl.BlockSpec(memory_space=pltpu.SMEM),
        ],
        out_specs=pl.BlockSpec(memory_space=pltpu.VMEM),
        scratch_shapes=[
            pltpu.VMEM((_KP, _KP), f32),
            pltpu.VMEM((_R, _L), f32),
            pltpu.VMEM((_R, _L), f32),
            pltpu.VMEM((_R, 1), f32),
        ],
    )(s2d, x1, y1, x2, y2, k2d)
    return out[:_K, :5]


# speedup vs baseline: 23.4761x; 1.0123x over previous
import jax
import jax.numpy as jnp
import numpy as np
from jax.experimental import pallas as pl
from jax.experimental.pallas import tpu as pltpu

_N = 20000
_R = 160
_L = 128
_NP = _R * _L
_K = 1000
_KP = 1024
_NEG = -1e9
_SENT = -3.0e38

_BITS = [b if (b := 1 << i) < 2**31 else b - 2**32 for i in range(32)]
_SIGN = -(2**31)


def _t(m):
    n = m.shape[1]
    i0 = jax.lax.broadcasted_iota(jnp.int32, (n, n), 0)
    i1 = jax.lax.broadcasted_iota(jnp.int32, (n, n), 1)
    eye = (i0 == i1).astype(jnp.float32)
    return jax.lax.dot_general(
        eye, m, (((1,), (1,)), ((), ())), preferred_element_type=jnp.float32, precision=jax.lax.Precision.HIGHEST)


def _nms_body(s_ref, b3_ref, k_ref, out_ref,
              iou_ref, msk_ref, pos_ref, off_ref):
    f32 = jnp.float32

    s = s_ref[...]
    sig = jax.nn.sigmoid(s)
    masked = jnp.where(sig >= 0.1, s, f32(_NEG))
    msk_ref[...] = masked

    ibits = jax.lax.bitcast_convert_type(masked, jnp.int32)
    skey = jnp.where(ibits < 0, ibits ^ jnp.int32(0x7FFFFFFF), ibits)

    sign = jnp.int32(_SIGN)
    uT = jnp.int32(0)
    for b in range(31, -1, -1):
        cand_u = uT | jnp.int32(_BITS[b])
        cand_s = cand_u ^ sign
        cnt = jnp.sum((skey >= cand_s).astype(jnp.int32))
        uT = jnp.where(cnt >= _K, cand_u, uT)
    T = uT ^ sign

    gt = skey > T
    eq = skey == T
    need = (_K - jnp.sum(gt.astype(jnp.int32))).astype(f32)

    li = jax.lax.broadcasted_iota(jnp.int32, (_L, _L), 0)
    lj = jax.lax.broadcasted_iota(jnp.int32, (_L, _L), 1)
    tri_u = (li <= lj).astype(f32)
    ri = jax.lax.broadcasted_iota(jnp.int32, (_R, _R), 0)
    rj = jax.lax.broadcasted_iota(jnp.int32, (_R, _R), 1)
    tri_l = (rj < ri).astype(f32)

    def excl_cumsum(m):
        incl = jnp.dot(m, tri_u, preferred_element_type=f32, precision=jax.lax.Precision.HIGHEST)
        rs = jnp.sum(m, axis=1, keepdims=True)
        off = jnp.dot(tri_l, rs, preferred_element_type=f32, precision=jax.lax.Precision.HIGHEST)
        return incl - m + off, off

    eqrank, _ = excl_cumsum(eq.astype(f32))
    sel = jnp.logical_or(gt, jnp.logical_and(eq, eqrank < need))
    pos, off = excl_cumsum(sel.astype(f32))
    pos_ref[...] = jnp.where(sel, pos, f32(-1.0))
    off_ref[...] = off

    col_kp_f = jax.lax.broadcasted_iota(jnp.int32, (_KP, 1), 0).astype(f32)
    lane_kp_row = jax.lax.broadcasted_iota(jnp.int32, (1, _KP), 1).astype(f32)
    lane_row_f = jax.lax.broadcasted_iota(jnp.int32, (1, _L), 1).astype(f32)
    zrow = jnp.zeros((2, _L), f32)

    col_l_f = jax.lax.broadcasted_iota(jnp.int32, (_L, 1), 0).astype(f32)
    zpad = jnp.zeros((8, _KP), f32)

    def comp_one(i, accT):
        start = off_ref[pl.ds(i, 1), :]
        relpos = pos_ref[pl.ds(i, 1), :] - start[0, 0]
        onehot = (col_l_f == relpos).astype(f32)
        idx_row = lane_row_f + i.astype(f32) * f32(_L)
        coords = jnp.reshape(b3_ref[pl.ds(i, 1), :, :], (4, _L))
        rows = jnp.concatenate(
            [msk_ref[pl.ds(i, 1), :], idx_row, coords, zrow], axis=0)
        contrib = jax.lax.dot_general(
            rows, onehot, (((1,), (1,)), ((), ())),
            preferred_element_type=f32, precision=jax.lax.Precision.HIGHEST)
        padded = jnp.concatenate([contrib, zpad], axis=1)
        rolled = pltpu.roll(padded, start[0, 0].astype(jnp.int32), axis=1)
        return accT + rolled

    def comp_body(i, accT):
        base = i * 4
        for u in range(4):
            accT = comp_one(base + u, accT)
        return accT

    accT = jax.lax.fori_loop(0, _R // 4, comp_body,
                             jnp.zeros((8, _KP + _L), f32))
    acc = _t(accT[:, :_KP])

    row_kp_f = jax.lax.broadcasted_iota(jnp.int32, (_KP, 1), 0).astype(f32)
    validrow = row_kp_f < f32(_K)
    rawc = jnp.where(validrow, acc[:, 0:1], f32(_SENT))
    idxc = jnp.where(validrow, acc[:, 1:2], f32(1e6) + row_kp_f)
    rawr = _t(rawc)
    idxr = _t(idxc)
    r_row = jnp.sum(((rawc > rawr).astype(f32)
                     + jnp.logical_and(rawc == rawr, idxc < idxr).astype(f32)),
                    axis=0, keepdims=True)
    perm = (col_kp_f == r_row).astype(f32)
    accadj = jnp.concatenate([rawc, idxc, acc[:, 2:8]], axis=1)
    srt = jnp.dot(perm, accadj, preferred_element_type=f32, precision=jax.lax.Precision.HIGHEST)
    srt_t = _t(srt)

    x1c, y1c = srt[:, 2:3], srt[:, 3:4]
    x2c, y2c = srt[:, 4:5], srt[:, 5:6]
    x1r, y1r = srt_t[2:3, :], srt_t[3:4, :]
    x2r, y2r = srt_t[4:5, :], srt_t[5:6, :]
    areac = (x2c - x1c) * (y2c - y1c)
    arear = (x2r - x1r) * (y2r - y1r)
    iw = jnp.maximum(jnp.minimum(x2c, x2r) - jnp.maximum(x1c, x1r), f32(0.0))
    ih = jnp.maximum(jnp.minimum(y2c, y2r) - jnp.maximum(y1c, y1r), f32(0.0))
    inter = iw * ih
    union = areac + arear - inter
    iou_ref[...] = inter / jnp.maximum(union, f32(1e-8))

    lane_kp_i = jax.lax.broadcasted_iota(jnp.int32, (1, _KP), 1)
    lane_kp_f = lane_kp_i.astype(f32)
    raw_row = srt_t[0:1, :]
    kval = k_ref[0, 0].astype(f32)
    keep0 = jnp.logical_and(raw_row > f32(-1e8), lane_kp_f < kval).astype(f32)

    nb = _KP // _L
    blocks = []
    for b in range(nb):
        lo = b * _L
        kb = keep0[:, lo:lo + _L]
        if b > 0:
            prev = jnp.concatenate(
                blocks + [jnp.zeros((1, _KP - lo), f32)], axis=1)
            adj = (iou_ref[:, lo:lo + _L] > f32(0.5)).astype(f32)
            supc = jnp.dot(prev, adj, preferred_element_type=f32,
                           precision=jax.lax.Precision.HIGHEST)
            kb = kb * (supc == f32(0.0)).astype(f32)

        blk = iou_ref[lo:lo + _L, lo:lo + _L]
        a_ut = jnp.logical_and(blk > f32(0.5), li < lj).astype(f32)
        valid_b = kb

        def fp_cond(c):
            return c[1]

        def fp_body(c):
            x, _ = c
            supn = jnp.dot(x, a_ut, preferred_element_type=f32,
                           precision=jax.lax.Precision.HIGHEST)
            new = valid_b * (supn == f32(0.0)).astype(f32)
            return (new, jnp.any(new != x))

        kb, _ = jax.lax.while_loop(fp_cond, fp_body, (kb, True))
        blocks.append(kb)
    keep = jnp.concatenate(blocks, axis=1)
    keepc = _t(keep)

    sigc = jax.nn.sigmoid(srt[:, 0:1])
    out = jnp.concatenate([srt[:, 2:6], sigc, jnp.zeros((_KP, 3), f32)],
                          axis=1)
    out_ref[...] = out * keepc


def kernel(boxes, scores, k):
    f32 = jnp.float32
    pad = _NP - _N
    s2d = jnp.concatenate(
        [scores.astype(f32), jnp.full((pad,), _NEG, f32)]).reshape(_R, _L)
    bx = jnp.concatenate([boxes.astype(f32), jnp.zeros((pad, 4), f32)], axis=0)
    b3 = bx.reshape(_R, _L, 4).transpose(0, 2, 1)
    k2d = jnp.asarray(k, jnp.int32).reshape(1, 1)

    out = pl.pallas_call(
        _nms_body,
        out_shape=jax.ShapeDtypeStruct((_KP, 8), f32),
        in_specs=[
            pl.BlockSpec(memory_space=pltpu.VMEM),
            pl.BlockSpec(memory_space=pltpu.VMEM),
            pl.BlockSpec(memory_space=pltpu.SMEM),
        ],
        out_specs=pl.BlockSpec(memory_space=pltpu.VMEM),
        scratch_shapes=[
            pltpu.VMEM((_KP, _KP), f32),
            pltpu.VMEM((_R, _L), f32),
            pltpu.VMEM((_R, _L), f32),
            pltpu.VMEM((_R, 1), f32),
        ],
    )(s2d, b3, k2d)
    return out[:_K, :5]


# speedup vs baseline: 23.4818x; 1.0002x over previous
import jax
import jax.numpy as jnp
from jax.experimental import pallas as pl
from jax.experimental.pallas import tpu as pltpu

_N = 20000
_R = 160
_L = 128
_NP = _R * _L
_K = 1000
_KP = 1024
_NEG = -1e9
_SENT = -3.0e38

_BITS = [b if (b := 1 << i) < 2**31 else b - 2**32 for i in range(32)]
_SIGN = -(2**31)


def _t(m):
    n = m.shape[1]
    i0 = jax.lax.broadcasted_iota(jnp.int32, (n, n), 0)
    i1 = jax.lax.broadcasted_iota(jnp.int32, (n, n), 1)
    eye = (i0 == i1).astype(jnp.float32)
    return jax.lax.dot_general(
        eye, m, (((1,), (1,)), ((), ())), preferred_element_type=jnp.float32, precision=jax.lax.Precision.HIGHEST)


def _nms_body(s_ref, b3_ref, k_ref, out_ref,
              iou_ref, msk_ref, pos_ref, off_ref):
    f32 = jnp.float32

    s = s_ref[...]
    sig = jax.nn.sigmoid(s)
    masked = jnp.where(sig >= 0.1, s, f32(_NEG))
    msk_ref[...] = masked

    ibits = jax.lax.bitcast_convert_type(masked, jnp.int32)
    skey = jnp.where(ibits < 0, ibits ^ jnp.int32(0x7FFFFFFF), ibits)

    sign = jnp.int32(_SIGN)
    uT = jnp.int32(0)
    for b in range(31, -1, -1):
        cand_u = uT | jnp.int32(_BITS[b])
        cand_s = cand_u ^ sign
        cnt = jnp.sum((skey >= cand_s).astype(jnp.int32))
        uT = jnp.where(cnt >= _K, cand_u, uT)
    T = uT ^ sign

    gt = skey > T
    eq = skey == T
    need = (_K - jnp.sum(gt.astype(jnp.int32))).astype(f32)

    li = jax.lax.broadcasted_iota(jnp.int32, (_L, _L), 0)
    lj = jax.lax.broadcasted_iota(jnp.int32, (_L, _L), 1)
    tri_u = (li <= lj).astype(f32)
    ri = jax.lax.broadcasted_iota(jnp.int32, (_R, _R), 0)
    rj = jax.lax.broadcasted_iota(jnp.int32, (_R, _R), 1)
    tri_l = (rj < ri).astype(f32)

    def excl_cumsum(m):
        incl = jnp.dot(m, tri_u, preferred_element_type=f32, precision=jax.lax.Precision.HIGHEST)
        rs = jnp.sum(m, axis=1, keepdims=True)
        off = jnp.dot(tri_l, rs, preferred_element_type=f32, precision=jax.lax.Precision.HIGHEST)
        return incl - m + off, off

    eqrank, _ = excl_cumsum(eq.astype(f32))
    sel = jnp.logical_or(gt, jnp.logical_and(eq, eqrank < need))
    pos, off = excl_cumsum(sel.astype(f32))
    pos_ref[...] = jnp.where(sel, pos, f32(-1.0))
    off_ref[...] = off

    col_kp_f = jax.lax.broadcasted_iota(jnp.int32, (_KP, 1), 0).astype(f32)
    lane_row_f = jax.lax.broadcasted_iota(jnp.int32, (1, _L), 1).astype(f32)
    zrow = jnp.zeros((2, _L), f32)

    col_l_f = jax.lax.broadcasted_iota(jnp.int32, (_L, 1), 0).astype(f32)
    zpad = jnp.zeros((8, _KP), f32)

    def comp_one(i, accT):
        start = off_ref[pl.ds(i, 1), :]
        relpos = pos_ref[pl.ds(i, 1), :] - start[0, 0]
        onehot = (col_l_f == relpos).astype(f32)
        idx_row = lane_row_f + i.astype(f32) * f32(_L)
        coords = jnp.reshape(b3_ref[pl.ds(i, 1), :, :], (4, _L))
        rows = jnp.concatenate(
            [msk_ref[pl.ds(i, 1), :], idx_row, coords, zrow], axis=0)
        contrib = jax.lax.dot_general(
            rows, onehot, (((1,), (1,)), ((), ())),
            preferred_element_type=f32, precision=jax.lax.Precision.HIGHEST)
        padded = jnp.concatenate([contrib, zpad], axis=1)
        rolled = pltpu.roll(padded, start[0, 0].astype(jnp.int32), axis=1)
        return accT + rolled

    def comp_body(i, accT):
        base = i * 4
        for u in range(4):
            accT = comp_one(base + u, accT)
        return accT

    accT = jax.lax.fori_loop(0, _R // 4, comp_body,
                             jnp.zeros((8, _KP + _L), f32))
    acc = _t(accT[:, :_KP])

    row_kp_f = jax.lax.broadcasted_iota(jnp.int32, (_KP, 1), 0).astype(f32)
    validrow = row_kp_f < f32(_K)
    rawc = jnp.where(validrow, acc[:, 0:1], f32(_SENT))
    idxc = jnp.where(validrow, acc[:, 1:2], f32(1e6) + row_kp_f)
    rawr = _t(rawc)
    idxr = _t(idxc)
    r_row = jnp.sum(((rawc > rawr).astype(f32)
                     + jnp.logical_and(rawc == rawr, idxc < idxr).astype(f32)),
                    axis=0, keepdims=True)
    perm = (col_kp_f == r_row).astype(f32)
    accadj = jnp.concatenate([rawc, idxc, acc[:, 2:8]], axis=1)
    srt = jnp.dot(perm, accadj, preferred_element_type=f32, precision=jax.lax.Precision.HIGHEST)
    srt_t = _t(srt)

    x1c, y1c = srt[:, 2:3], srt[:, 3:4]
    x2c, y2c = srt[:, 4:5], srt[:, 5:6]
    x1r, y1r = srt_t[2:3, :], srt_t[3:4, :]
    x2r, y2r = srt_t[4:5, :], srt_t[5:6, :]
    areac = (x2c - x1c) * (y2c - y1c)
    arear = (x2r - x1r) * (y2r - y1r)
    iw = jnp.maximum(jnp.minimum(x2c, x2r) - jnp.maximum(x1c, x1r), f32(0.0))
    ih = jnp.maximum(jnp.minimum(y2c, y2r) - jnp.maximum(y1c, y1r), f32(0.0))
    inter = iw * ih
    union = areac + arear - inter
    iou_ref[...] = inter / jnp.maximum(union, f32(1e-8))

    lane_kp_i = jax.lax.broadcasted_iota(jnp.int32, (1, _KP), 1)
    lane_kp_f = lane_kp_i.astype(f32)
    raw_row = srt_t[0:1, :]
    kval = k_ref[0, 0].astype(f32)
    keep0 = jnp.logical_and(raw_row > f32(-1e8), lane_kp_f < kval).astype(f32)

    nb = _KP // _L
    blocks = []
    for b in range(nb):
        lo = b * _L
        kb = keep0[:, lo:lo + _L]
        if b > 0:
            prev = jnp.concatenate(
                blocks + [jnp.zeros((1, _KP - lo), f32)], axis=1)
            adj = (iou_ref[:, lo:lo + _L] > f32(0.5)).astype(f32)
            supc = jnp.dot(prev, adj, preferred_element_type=f32,
                           precision=jax.lax.Precision.HIGHEST)
            kb = kb * (supc == f32(0.0)).astype(f32)

        blk = iou_ref[lo:lo + _L, lo:lo + _L]
        a_ut = jnp.logical_and(blk > f32(0.5), li < lj).astype(f32)
        valid_b = kb

        def fp_cond(c):
            return c[1]

        def fp_body(c):
            x, _ = c
            supn = jnp.dot(x, a_ut, preferred_element_type=f32,
                           precision=jax.lax.Precision.HIGHEST)
            new = valid_b * (supn == f32(0.0)).astype(f32)
            return (new, jnp.any(new != x))

        kb, _ = jax.lax.while_loop(fp_cond, fp_body, (kb, True))
        blocks.append(kb)
    keep = jnp.concatenate(blocks, axis=1)
    keepc = _t(keep)

    sigc = jax.nn.sigmoid(srt[:, 0:1])
    out = jnp.concatenate([srt[:, 2:6], sigc, jnp.zeros((_KP, 3), f32)],
                          axis=1)
    out_ref[...] = out * keepc


def kernel(boxes, scores, k):
    f32 = jnp.float32
    pad = _NP - _N
    s2d = jnp.concatenate(
        [scores.astype(f32), jnp.full((pad,), _NEG, f32)]).reshape(_R, _L)
    bx = jnp.concatenate([boxes.astype(f32), jnp.zeros((pad, 4), f32)], axis=0)
    b3 = bx.reshape(_R, _L, 4).transpose(0, 2, 1)
    k2d = jnp.asarray(k, jnp.int32).reshape(1, 1)

    out = pl.pallas_call(
        _nms_body,
        out_shape=jax.ShapeDtypeStruct((_KP, 8), f32),
        in_specs=[
            pl.BlockSpec(memory_space=pltpu.VMEM),
            pl.BlockSpec(memory_space=pltpu.VMEM),
            pl.BlockSpec(memory_space=pltpu.SMEM),
        ],
        out_specs=pl.BlockSpec(memory_space=pltpu.VMEM),
        scratch_shapes=[
            pltpu.VMEM((_KP, _KP), f32),
            pltpu.VMEM((_R, _L), f32),
            pltpu.VMEM((_R, _L), f32),
            pltpu.VMEM((_R, 1), f32),
        ],
    )(s2d, b3, k2d)
    return out[:_K, :5]
